# baseline (speedup 1.0000x reference)
import functools

import jax
import jax.numpy as jnp
from jax import lax
from jax.experimental import pallas as pl
from jax.experimental.pallas import tpu as pltpu
from jax.experimental.pallas import tpu_sc as plsc

_BT = 16 * 576
_D = 256
_S = 4
_K = 1024
_SUB = _D // _S
_BLK = 512

_NC, _NS = 2, 16
_NW = _NC * _NS
_TCH = 8
_BPW = _BT // _TCH
_IDXC = 128
_NSTRM = _BPW // _IDXC


def _dist_body(x_ref, wt_ref, idx_ref, err_ref):
    i = pl.program_id(0)

    @pl.when(i == 0)
    def _init():
        err_ref[0, 0] = 0.0

    xb = x_ref[...]
    acc = jnp.float32(0.0)
    for s in range(_S):
        xi = xb[:, s * _SUB:(s + 1) * _SUB]
        wt2 = wt_ref[s]
        xnorm = jnp.sum(xi * xi, axis=1, keepdims=True)
        cbnorm = jnp.sum(wt2 * wt2, axis=0, keepdims=True) * 0.25
        scores2 = jnp.dot(xi, wt2, preferred_element_type=jnp.float32)
        d = (xnorm + cbnorm) + scores2
        m = jnp.min(d, axis=1, keepdims=True)
        iota = lax.broadcasted_iota(jnp.int32, d.shape, 1)
        sel = jnp.where(d == m, iota, jnp.int32(2 ** 30))
        idx = jnp.min(sel, axis=1).astype(jnp.int32)
        idx_ref[s, :] = idx + s * _K
        acc = acc + jnp.sum(m)
    err_ref[0, 0] += acc * (1.25 / (_BT * _SUB))


def _distances(xf, wt):
    return pl.pallas_call(
        _dist_body,
        grid=(_BT // _BLK,),
        in_specs=[
            pl.BlockSpec((_BLK, _D), lambda i: (i, 0)),
            pl.BlockSpec((_S, _SUB, _K), lambda i: (0, 0, 0)),
        ],
        out_specs=[
            pl.BlockSpec((_S, _BLK), lambda i: (0, i)),
            pl.BlockSpec(memory_space=pltpu.SMEM),
        ],
        out_shape=[
            jax.ShapeDtypeStruct((_S, _BT), jnp.int32),
            jax.ShapeDtypeStruct((1, 1), jnp.float32),
        ],
    )(xf, wt)


def _gather_body(table_ref, idx_ref, out_ref, idx_v, rows_v, sem):
    wid = lax.axis_index("s") * _NC + lax.axis_index("c")
    split = wid // _TCH
    tchunk = lax.rem(wid, _TCH)
    base = pl.multiple_of(wid * _BPW, _IDXC)
    pltpu.sync_copy(idx_ref.at[pl.ds(base, _BPW)], idx_v)
    copies = []
    for j in range(_NSTRM):
        copies.append(pltpu.async_copy(
            table_ref.at[idx_v.at[pl.ds(j * _IDXC, _IDXC)]],
            rows_v.at[pl.ds(j * _IDXC, _IDXC)],
            sem,
        ))
    for cp in copies:
        cp.wait()
    tbase = pl.multiple_of(tchunk * _BPW, _IDXC)
    pltpu.sync_copy(rows_v, out_ref.at[split, pl.ds(tbase, _BPW)])


@functools.lru_cache(maxsize=1)
def _gather_kernel():
    return pl.kernel(
        _gather_body,
        out_type=jax.ShapeDtypeStruct((_S, _BT, _SUB), jnp.float32),
        mesh=plsc.VectorSubcoreMesh(
            core_axis_name="c", subcore_axis_name="s",
            num_cores=_NC, num_subcores=_NS),
        scratch_types=[
            pltpu.VMEM((_BPW,), jnp.int32),
            pltpu.VMEM((_BPW, _SUB), jnp.float32),
            pltpu.SemaphoreType.DMA,
        ],
        compiler_params=pltpu.CompilerParams(use_tc_tiling_on_sc=False),
    )


def kernel(x, W):
    B, T, D = x.shape
    xf = x.reshape(B * T, D)
    wt2 = W.transpose(0, 2, 1) * jnp.float32(-2.0)
    idxg, err = _distances(xf, wt2)
    table = W.reshape(_S * _K, _SUB)
    idx_flat = idxg.reshape(_S * _BT)
    quant = _gather_kernel()(table, idx_flat)
    quant = quant.transpose(1, 0, 2).reshape(B, T, D)
    return quant, err[0, 0]


# speedup vs baseline: 1.2342x; 1.2342x over previous
import functools

import jax
import jax.numpy as jnp
from jax import lax
from jax.experimental import pallas as pl
from jax.experimental.pallas import tpu as pltpu
from jax.experimental.pallas import tpu_sc as plsc

_B = 16
_T = 576
_BT = _B * _T
_D = 256
_S = 4
_K = 1024
_SUB = _D // _S

_NC, _NS = 2, 16
_NW = _NC * _NS
_UPW = (_B * _S) // _NW
_IDXC = 128


def _dist_body(x_ref, wt_ref, idx_ref, err_ref):
    i = pl.program_id(0)

    @pl.when(i == 0)
    def _init():
        err_ref[0, 0] = 0.0

    xb = x_ref[0]
    acc = jnp.float32(0.0)
    for s in range(_S):
        xi = xb[:, s * _SUB:(s + 1) * _SUB]
        wt2 = wt_ref[s]
        xnorm = jnp.sum(xi * xi, axis=1, keepdims=True)
        cbnorm = jnp.sum(wt2 * wt2, axis=0, keepdims=True) * 0.25
        scores2 = jnp.dot(xi, wt2, preferred_element_type=jnp.float32)
        d = (xnorm + cbnorm) + scores2
        m = jnp.min(d, axis=1, keepdims=True)
        iota = lax.broadcasted_iota(jnp.int32, d.shape, 1)
        sel = jnp.where(d == m, iota, jnp.int32(2 ** 30))
        idx = jnp.min(sel, axis=1).astype(jnp.int32)
        idx_ref[0, s, :] = idx + s * _K
        acc = acc + jnp.sum(m)
    err_ref[0, 0] += acc * (1.25 / (_BT * _SUB))


def _distances(x, wt):
    return pl.pallas_call(
        _dist_body,
        grid=(_B,),
        in_specs=[
            pl.BlockSpec((1, _T, _D), lambda i: (i, 0, 0)),
            pl.BlockSpec((_S, _SUB, _K), lambda i: (0, 0, 0)),
        ],
        out_specs=[
            pl.BlockSpec((1, _S, _T), lambda i: (i, 0, 0)),
            pl.BlockSpec(memory_space=pltpu.SMEM),
        ],
        out_shape=[
            jax.ShapeDtypeStruct((_B, _S, _T), jnp.int32),
            jax.ShapeDtypeStruct((1, 1), jnp.float32),
        ],
    )(x, wt)


def _gather_body(table_ref, idx_ref, out_ref, idx_v, rows_v, sem):
    wid = lax.axis_index("s") * _NC + lax.axis_index("c")
    for u in range(_UPW):
        unit = wid * _UPW + u
        b = unit // _S
        s = lax.rem(unit, _S)
        base = pl.multiple_of(unit * _T, 8)
        pltpu.sync_copy(idx_ref.at[pl.ds(base, _T)], idx_v)
        copies = []
        off = 0
        while off < _T:
            n = min(_IDXC, _T - off)
            copies.append(pltpu.async_copy(
                table_ref.at[idx_v.at[pl.ds(off, n)]],
                rows_v.at[pl.ds(off, n)],
                sem,
            ))
            off += n
        for cp in copies:
            cp.wait()
        tbase = pl.multiple_of(b * _T, 8)
        sbase = pl.multiple_of(s * _SUB, 8)
        pltpu.sync_copy(
            rows_v, out_ref.at[pl.ds(tbase, _T), pl.ds(sbase, _SUB)])


@functools.lru_cache(maxsize=1)
def _gather_kernel():
    return pl.kernel(
        _gather_body,
        out_type=jax.ShapeDtypeStruct((_BT, _D), jnp.float32),
        mesh=plsc.VectorSubcoreMesh(
            core_axis_name="c", subcore_axis_name="s",
            num_cores=_NC, num_subcores=_NS),
        scratch_types=[
            pltpu.VMEM((_T,), jnp.int32),
            pltpu.VMEM((_T, _SUB), jnp.float32),
            pltpu.SemaphoreType.DMA,
        ],
        compiler_params=pltpu.CompilerParams(use_tc_tiling_on_sc=False),
    )


def kernel(x, W):
    B, T, D = x.shape
    wt2 = W.transpose(0, 2, 1) * jnp.float32(-2.0)
    idxg, err = _distances(x, wt2)
    table = W.reshape(_S * _K, _SUB)
    idx_flat = idxg.reshape(_B * _S * _T)
    quant = _gather_kernel()(table, idx_flat)
    return quant.reshape(B, T, D), err[0, 0]


# speedup vs baseline: 1.3897x; 1.1259x over previous
import functools

import jax
import jax.numpy as jnp
from jax import lax
from jax.experimental import pallas as pl
from jax.experimental.pallas import tpu as pltpu
from jax.experimental.pallas import tpu_sc as plsc

_B = 16
_T = 576
_BT = _B * _T
_D = 256
_S = 4
_K = 1024
_SUB = _D // _S

_NC, _NS = 2, 16
_NW = _NC * _NS
_UPW = (_B * _S) // _NW
_IDXC = 128


def _dist_body(x_ref, wt_ref, idx_ref, err_ref):
    i = pl.program_id(0)

    @pl.when(i == 0)
    def _init():
        err_ref[0, 0] = 0.0

    xb = x_ref[0]
    acc = jnp.float32(0.0)
    iota1 = lax.broadcasted_iota(jnp.int32, (1, _K), 1).astype(jnp.float32)
    for s in range(_S):
        xi = xb[:, s * _SUB:(s + 1) * _SUB]
        wt2 = wt_ref[s]
        xnorm = jnp.sum(xi * xi, axis=1, keepdims=True)
        cbnorm = jnp.sum(wt2 * wt2, axis=0, keepdims=True) * 0.25
        scores2 = jnp.dot(xi, wt2, preferred_element_type=jnp.float32)
        d = (xnorm + cbnorm) + scores2
        m = jnp.min(d, axis=1, keepdims=True)
        sel = jnp.where(d == m, iota1, jnp.float32(2.0 * _K))
        idxf = jnp.min(sel, axis=1, keepdims=True)
        idx_ref[0, s, :, :] = idxf.astype(jnp.int32) + s * _K
        acc = acc + jnp.sum(m)
    err_ref[0, 0] += acc * (1.25 / (_BT * _SUB))


def _distances(x, wt):
    return pl.pallas_call(
        _dist_body,
        grid=(_B,),
        in_specs=[
            pl.BlockSpec((1, _T, _D), lambda i: (i, 0, 0)),
            pl.BlockSpec((_S, _SUB, _K), lambda i: (0, 0, 0)),
        ],
        out_specs=[
            pl.BlockSpec((1, _S, _T, 1), lambda i: (i, 0, 0, 0)),
            pl.BlockSpec(memory_space=pltpu.SMEM),
        ],
        out_shape=[
            jax.ShapeDtypeStruct((_B, _S, _T, 1), jnp.int32),
            jax.ShapeDtypeStruct((1, 1), jnp.float32),
        ],
    )(x, wt)


def _gather_body(table_ref, idx_ref, out_ref, idx_v, rows_v, sem):
    wid = lax.axis_index("s") * _NC + lax.axis_index("c")
    for u in range(_UPW):
        unit = wid * _UPW + u
        b = unit // _S
        s = lax.rem(unit, _S)
        base = pl.multiple_of(unit * _T, 8)
        pltpu.sync_copy(idx_ref.at[pl.ds(base, _T)], idx_v)
        copies = []
        off = 0
        while off < _T:
            n = min(_IDXC, _T - off)
            copies.append(pltpu.async_copy(
                table_ref.at[idx_v.at[pl.ds(off, n)]],
                rows_v.at[pl.ds(off, n)],
                sem,
            ))
            off += n
        for cp in copies:
            cp.wait()
        tbase = pl.multiple_of(b * _T, 8)
        sbase = pl.multiple_of(s * _SUB, 8)
        pltpu.sync_copy(
            rows_v, out_ref.at[pl.ds(tbase, _T), pl.ds(sbase, _SUB)])


@functools.lru_cache(maxsize=1)
def _gather_kernel():
    return pl.kernel(
        _gather_body,
        out_type=jax.ShapeDtypeStruct((_BT, _D), jnp.float32),
        mesh=plsc.VectorSubcoreMesh(
            core_axis_name="c", subcore_axis_name="s",
            num_cores=_NC, num_subcores=_NS),
        scratch_types=[
            pltpu.VMEM((_T,), jnp.int32),
            pltpu.VMEM((_T, _SUB), jnp.float32),
            pltpu.SemaphoreType.DMA,
        ],
        compiler_params=pltpu.CompilerParams(use_tc_tiling_on_sc=False),
    )


def kernel(x, W):
    B, T, D = x.shape
    wt2 = W.transpose(0, 2, 1) * jnp.float32(-2.0)
    idxg, err = _distances(x, wt2)
    table = W.reshape(_S * _K, _SUB)
    idx_flat = idxg.reshape(_B * _S * _T)
    quant = _gather_kernel()(table, idx_flat)
    return quant.reshape(B, T, D), err[0, 0]


# speedup vs baseline: 1.4311x; 1.0298x over previous
import functools

import jax
import jax.numpy as jnp
from jax import lax
from jax.experimental import pallas as pl
from jax.experimental.pallas import tpu as pltpu
from jax.experimental.pallas import tpu_sc as plsc

_B = 16
_T = 576
_BT = _B * _T
_D = 256
_S = 4
_K = 1024
_SUB = _D // _S

_NC, _NS = 2, 16
_NW = _NC * _NS
_UPW = (_B * _S) // _NW
_IDXC = 128


_RPB = 2


def _dist_body(x_ref, wt_ref, idx_ref, err_ref):
    i = pl.program_id(0)

    @pl.when(i == 0)
    def _init():
        err_ref[0, 0] = 0.0

    acc = jnp.float32(0.0)
    iota1 = lax.broadcasted_iota(jnp.int32, (1, _K), 1).astype(jnp.float32)
    for r in range(_RPB):
        xb = x_ref[r]
        for s in range(_S):
            xi = xb[:, s * _SUB:(s + 1) * _SUB]
            wt2 = wt_ref[s]
            xnorm = jnp.sum(xi * xi, axis=1, keepdims=True)
            cbnorm = jnp.sum(wt2 * wt2, axis=0, keepdims=True) * 0.25
            scores2 = jnp.dot(xi, wt2, preferred_element_type=jnp.float32)
            d = (xnorm + cbnorm) + scores2
            m = jnp.min(d, axis=1, keepdims=True)
            sel = jnp.where(d == m, iota1, jnp.float32(2.0 * _K))
            idxf = jnp.min(sel, axis=1, keepdims=True)
            idx_ref[r, s, :, :] = idxf.astype(jnp.int32) + s * _K
            acc = acc + jnp.sum(m)
    err_ref[0, 0] += acc * (1.25 / (_BT * _SUB))


def _distances(x, wt):
    return pl.pallas_call(
        _dist_body,
        grid=(_B // _RPB,),
        in_specs=[
            pl.BlockSpec((_RPB, _T, _D), lambda i: (i, 0, 0)),
            pl.BlockSpec((_S, _SUB, _K), lambda i: (0, 0, 0)),
        ],
        out_specs=[
            pl.BlockSpec((_RPB, _S, _T, 1), lambda i: (i, 0, 0, 0)),
            pl.BlockSpec(memory_space=pltpu.SMEM),
        ],
        out_shape=[
            jax.ShapeDtypeStruct((_B, _S, _T, 1), jnp.int32),
            jax.ShapeDtypeStruct((1, 1), jnp.float32),
        ],
    )(x, wt)


def _gather_body(table_ref, idx_ref, out_ref, idx_v, rows_v, sem):
    wid = lax.axis_index("s") * _NC + lax.axis_index("c")
    for u in range(_UPW):
        unit = wid * _UPW + u
        b = unit // _S
        s = lax.rem(unit, _S)
        base = pl.multiple_of(unit * _T, 8)
        pltpu.sync_copy(idx_ref.at[pl.ds(base, _T)], idx_v)
        copies = []
        off = 0
        while off < _T:
            n = min(_IDXC, _T - off)
            copies.append(pltpu.async_copy(
                table_ref.at[idx_v.at[pl.ds(off, n)]],
                rows_v.at[pl.ds(off, n)],
                sem,
            ))
            off += n
        for cp in copies:
            cp.wait()
        tbase = pl.multiple_of(b * _T, 8)
        sbase = pl.multiple_of(s * _SUB, 8)
        pltpu.sync_copy(
            rows_v, out_ref.at[pl.ds(tbase, _T), pl.ds(sbase, _SUB)])


@functools.lru_cache(maxsize=1)
def _gather_kernel():
    return pl.kernel(
        _gather_body,
        out_type=jax.ShapeDtypeStruct((_BT, _D), jnp.float32),
        mesh=plsc.VectorSubcoreMesh(
            core_axis_name="c", subcore_axis_name="s",
            num_cores=_NC, num_subcores=_NS),
        scratch_types=[
            pltpu.VMEM((_T,), jnp.int32),
            pltpu.VMEM((_T, _SUB), jnp.float32),
            pltpu.SemaphoreType.DMA,
        ],
        compiler_params=pltpu.CompilerParams(use_tc_tiling_on_sc=False),
    )


def kernel(x, W):
    B, T, D = x.shape
    wt2 = W.transpose(0, 2, 1) * jnp.float32(-2.0)
    idxg, err = _distances(x, wt2)
    table = W.reshape(_S * _K, _SUB)
    idx_flat = idxg.reshape(_B * _S * _T)
    quant = _gather_kernel()(table, idx_flat)
    return quant.reshape(B, T, D), err[0, 0]


# speedup vs baseline: 1.4558x; 1.0173x over previous
import functools

import jax
import jax.numpy as jnp
from jax import lax
from jax.experimental import pallas as pl
from jax.experimental.pallas import tpu as pltpu
from jax.experimental.pallas import tpu_sc as plsc

_B = 16
_T = 576
_BT = _B * _T
_D = 256
_S = 4
_K = 1024
_SUB = _D // _S

_NC, _NS = 2, 16
_NW = _NC * _NS
_UPW = (_B * _S) // _NW
_IDXC = 128


_RPB = 2


def _dist_body(x_ref, wt_ref, idx_ref, err_ref):
    i = pl.program_id(0)

    @pl.when(i == 0)
    def _init():
        err_ref[0, 0] = 0.0

    acc = jnp.float32(0.0)
    iota1 = lax.broadcasted_iota(jnp.int32, (1, _K), 1).astype(jnp.float32)
    for r in range(_RPB):
        xb = x_ref[r]
        for s in range(_S):
            xi = xb[:, s * _SUB:(s + 1) * _SUB]
            wt2 = wt_ref[s]
            xnorm = jnp.sum(xi * xi, axis=1, keepdims=True)
            cbnorm = jnp.sum(wt2 * wt2, axis=0, keepdims=True) * 0.25
            scores2 = jnp.dot(xi, wt2, preferred_element_type=jnp.float32)
            d = (xnorm + cbnorm) + scores2
            m = jnp.min(d, axis=1, keepdims=True)
            sel = jnp.where(d == m, iota1, jnp.float32(2.0 * _K))
            idxf = jnp.min(sel, axis=1, keepdims=True)
            idx_ref[r, s, :, :] = idxf.astype(jnp.int32) + s * _K
            acc = acc + jnp.sum(m)
    err_ref[0, 0] += acc * (1.25 / (_BT * _SUB))


def _distances(x, wt):
    return pl.pallas_call(
        _dist_body,
        grid=(_B // _RPB,),
        in_specs=[
            pl.BlockSpec((_RPB, _T, _D), lambda i: (i, 0, 0)),
            pl.BlockSpec((_S, _SUB, _K), lambda i: (0, 0, 0)),
        ],
        out_specs=[
            pl.BlockSpec((_RPB, _S, _T, 1), lambda i: (i, 0, 0, 0)),
            pl.BlockSpec(memory_space=pltpu.SMEM),
        ],
        out_shape=[
            jax.ShapeDtypeStruct((_B, _S, _T, 1), jnp.int32),
            jax.ShapeDtypeStruct((1, 1), jnp.float32),
        ],
    )(x, wt)


def _gather_body(table_ref, idx_ref, out_ref, idx_v, rows_v, sem):
    wid = lax.axis_index("s") * _NC + lax.axis_index("c")
    nw = _UPW * _T
    base = pl.multiple_of(wid * nw, 8)
    pltpu.sync_copy(idx_ref.at[pl.ds(base, nw)], idx_v)
    copies = []
    for off in range(0, nw, _IDXC):
        copies.append(pltpu.async_copy(
            table_ref.at[idx_v.at[pl.ds(off, _IDXC)]],
            rows_v.at[pl.ds(off, _IDXC)],
            sem,
        ))
    for cp in copies:
        cp.wait()
    for u in range(_UPW):
        unit = wid * _UPW + u
        b = unit // _S
        s = lax.rem(unit, _S)
        tbase = pl.multiple_of(b * _T, 8)
        sbase = pl.multiple_of(s * _SUB, 8)
        pltpu.sync_copy(
            rows_v.at[pl.ds(u * _T, _T)],
            out_ref.at[pl.ds(tbase, _T), pl.ds(sbase, _SUB)])


@functools.lru_cache(maxsize=1)
def _gather_kernel():
    return pl.kernel(
        _gather_body,
        out_type=jax.ShapeDtypeStruct((_BT, _D), jnp.float32),
        mesh=plsc.VectorSubcoreMesh(
            core_axis_name="c", subcore_axis_name="s",
            num_cores=_NC, num_subcores=_NS),
        scratch_types=[
            pltpu.VMEM((_UPW * _T,), jnp.int32),
            pltpu.VMEM((_UPW * _T, _SUB), jnp.float32),
            pltpu.SemaphoreType.DMA,
        ],
        compiler_params=pltpu.CompilerParams(use_tc_tiling_on_sc=False),
    )


def kernel(x, W):
    B, T, D = x.shape
    wt2 = W.transpose(0, 2, 1) * jnp.float32(-2.0)
    idxg, err = _distances(x, wt2)
    table = W.reshape(_S * _K, _SUB)
    idx_flat = idxg.reshape(_B * _S * _T)
    quant = _gather_kernel()(table, idx_flat)
    return quant.reshape(B, T, D), err[0, 0]


# speedup vs baseline: 1.4749x; 1.0131x over previous
import functools

import jax
import jax.numpy as jnp
from jax import lax
from jax.experimental import pallas as pl
from jax.experimental.pallas import tpu as pltpu
from jax.experimental.pallas import tpu_sc as plsc

_B = 16
_T = 576
_BT = _B * _T
_D = 256
_S = 4
_K = 1024
_SUB = _D // _S

_NC, _NS = 2, 16
_NW = _NC * _NS
_UPW = (_B * _S) // _NW
_IDXC = 128


_RPB = 2


def _dist_body(x_ref, w_ref, idx_ref, err_ref, wt_scr):
    i = pl.program_id(0)

    @pl.when(i == 0)
    def _init():
        err_ref[0, 0] = 0.0
        for s in range(_S):
            wt_scr[s] = jnp.transpose(w_ref[s]) * jnp.float32(-2.0)

    wt_ref = wt_scr

    acc = jnp.float32(0.0)
    iota1 = lax.broadcasted_iota(jnp.int32, (1, _K), 1).astype(jnp.float32)
    for r in range(_RPB):
        xb = x_ref[r]
        for s in range(_S):
            xi = xb[:, s * _SUB:(s + 1) * _SUB]
            wt2 = wt_ref[s]
            xnorm = jnp.sum(xi * xi, axis=1, keepdims=True)
            cbnorm = jnp.sum(wt2 * wt2, axis=0, keepdims=True) * 0.25
            scores2 = jnp.dot(xi, wt2, preferred_element_type=jnp.float32)
            d = (xnorm + cbnorm) + scores2
            m = jnp.min(d, axis=1, keepdims=True)
            sel = jnp.where(d == m, iota1, jnp.float32(2.0 * _K))
            idxf = jnp.min(sel, axis=1, keepdims=True)
            idx_ref[r, s, :, :] = idxf.astype(jnp.int32) + s * _K
            acc = acc + jnp.sum(m)
    err_ref[0, 0] += acc * (1.25 / (_BT * _SUB))


def _distances(x, wt):
    return pl.pallas_call(
        _dist_body,
        grid=(_B // _RPB,),
        in_specs=[
            pl.BlockSpec((_RPB, _T, _D), lambda i: (i, 0, 0)),
            pl.BlockSpec((_S, _K, _SUB), lambda i: (0, 0, 0)),
        ],
        scratch_shapes=[pltpu.VMEM((_S, _SUB, _K), jnp.float32)],
        out_specs=[
            pl.BlockSpec((_RPB, _S, _T, 1), lambda i: (i, 0, 0, 0)),
            pl.BlockSpec(memory_space=pltpu.SMEM),
        ],
        out_shape=[
            jax.ShapeDtypeStruct((_B, _S, _T, 1), jnp.int32),
            jax.ShapeDtypeStruct((1, 1), jnp.float32),
        ],
    )(x, wt)


def _gather_body(table_ref, idx_ref, out_ref, idx_v, rows_v, sem):
    wid = lax.axis_index("s") * _NC + lax.axis_index("c")
    nw = _UPW * _T
    base = pl.multiple_of(wid * nw, 8)
    pltpu.sync_copy(idx_ref.at[pl.ds(base, nw)], idx_v)
    copies = []
    for off in range(0, nw, _IDXC):
        copies.append(pltpu.async_copy(
            table_ref.at[idx_v.at[pl.ds(off, _IDXC)]],
            rows_v.at[pl.ds(off, _IDXC)],
            sem,
        ))
    for cp in copies:
        cp.wait()
    for u in range(_UPW):
        unit = wid * _UPW + u
        b = unit // _S
        s = lax.rem(unit, _S)
        tbase = pl.multiple_of(b * _T, 8)
        sbase = pl.multiple_of(s * _SUB, 8)
        pltpu.sync_copy(
            rows_v.at[pl.ds(u * _T, _T)],
            out_ref.at[pl.ds(tbase, _T), pl.ds(sbase, _SUB)])


@functools.lru_cache(maxsize=1)
def _gather_kernel():
    return pl.kernel(
        _gather_body,
        out_type=jax.ShapeDtypeStruct((_BT, _D), jnp.float32),
        mesh=plsc.VectorSubcoreMesh(
            core_axis_name="c", subcore_axis_name="s",
            num_cores=_NC, num_subcores=_NS),
        scratch_types=[
            pltpu.VMEM((_UPW * _T,), jnp.int32),
            pltpu.VMEM((_UPW * _T, _SUB), jnp.float32),
            pltpu.SemaphoreType.DMA,
        ],
        compiler_params=pltpu.CompilerParams(use_tc_tiling_on_sc=False),
    )


def kernel(x, W):
    B, T, D = x.shape
    idxg, err = _distances(x, W)
    table = W.reshape(_S * _K, _SUB)
    idx_flat = idxg.reshape(_B * _S * _T)
    quant = _gather_kernel()(table, idx_flat)
    return quant.reshape(B, T, D), err[0, 0]


# speedup vs baseline: 1.6210x; 1.0990x over previous
import functools

import jax
import jax.numpy as jnp
from jax import lax
from jax.experimental import pallas as pl
from jax.experimental.pallas import tpu as pltpu
from jax.experimental.pallas import tpu_sc as plsc

_B = 16
_T = 576
_BT = _B * _T
_D = 256
_S = 4
_K = 1024
_SUB = _D // _S

_NC, _NS = 2, 16
_NW = _NC * _NS
_UPW = (_B * _S) // _NW
_IDXC = 128


_RPB = 2


def _dist_body(x_ref, w_ref, idx_ref, err_ref, wt_scr):
    i = pl.program_id(0)

    @pl.when(i == 0)
    def _init():
        err_ref[0, 0] = 0.0
        for s in range(_S):
            wt_scr[s] = jnp.transpose(w_ref[s]) * jnp.float32(-2.0)

    wt_ref = wt_scr

    acc = jnp.float32(0.0)
    iota1 = lax.broadcasted_iota(jnp.int32, (1, _K), 1).astype(jnp.float32)
    cols = []
    for r in range(_RPB):
        xb = x_ref[r]
        for s in range(_S):
            xi = xb[:, s * _SUB:(s + 1) * _SUB]
            wt2 = wt_ref[s]
            xnorm = jnp.sum(xi * xi, axis=1, keepdims=True)
            cbnorm = jnp.sum(wt2 * wt2, axis=0, keepdims=True) * 0.25
            scores2 = jnp.dot(xi, wt2, preferred_element_type=jnp.float32)
            d = (xnorm + cbnorm) + scores2
            m = jnp.min(d, axis=1, keepdims=True)
            sel = jnp.where(d == m, iota1, jnp.float32(2.0 * _K))
            cols.append(jnp.min(sel, axis=1, keepdims=True))
            acc = acc + jnp.sum(m)
    mat = jnp.concatenate(cols, axis=1)
    idxr = jnp.transpose(mat)
    offs = (lax.broadcasted_iota(jnp.int32, (_RPB * _S, 1), 0) % _S) * _K
    idxi = idxr.astype(jnp.int32) + offs
    idx_ref[...] = idxi.reshape(_RPB, _S, _T)
    err_ref[0, 0] += acc * (1.25 / (_BT * _SUB))


def _distances(x, wt):
    return pl.pallas_call(
        _dist_body,
        grid=(_B // _RPB,),
        in_specs=[
            pl.BlockSpec((_RPB, _T, _D), lambda i: (i, 0, 0)),
            pl.BlockSpec((_S, _K, _SUB), lambda i: (0, 0, 0)),
        ],
        scratch_shapes=[pltpu.VMEM((_S, _SUB, _K), jnp.float32)],
        out_specs=[
            pl.BlockSpec((_RPB, _S, _T), lambda i: (i, 0, 0)),
            pl.BlockSpec(memory_space=pltpu.SMEM),
        ],
        out_shape=[
            jax.ShapeDtypeStruct((_B, _S, _T), jnp.int32),
            jax.ShapeDtypeStruct((1, 1), jnp.float32),
        ],
    )(x, wt)


def _gather_body(table_ref, idx_ref, out_ref, idx_v, rows_v, sem):
    wid = lax.axis_index("s") * _NC + lax.axis_index("c")
    nw = _UPW * _T
    base = pl.multiple_of(wid * nw, 8)
    pltpu.sync_copy(idx_ref.at[pl.ds(base, nw)], idx_v)
    copies = []
    for off in range(0, nw, _IDXC):
        copies.append(pltpu.async_copy(
            table_ref.at[idx_v.at[pl.ds(off, _IDXC)]],
            rows_v.at[pl.ds(off, _IDXC)],
            sem,
        ))
    for cp in copies:
        cp.wait()
    for u in range(_UPW):
        unit = wid * _UPW + u
        b = unit // _S
        s = lax.rem(unit, _S)
        tbase = pl.multiple_of(b * _T, 8)
        sbase = pl.multiple_of(s * _SUB, 8)
        pltpu.sync_copy(
            rows_v.at[pl.ds(u * _T, _T)],
            out_ref.at[pl.ds(tbase, _T), pl.ds(sbase, _SUB)])


@functools.lru_cache(maxsize=1)
def _gather_kernel():
    return pl.kernel(
        _gather_body,
        out_type=jax.ShapeDtypeStruct((_BT, _D), jnp.float32),
        mesh=plsc.VectorSubcoreMesh(
            core_axis_name="c", subcore_axis_name="s",
            num_cores=_NC, num_subcores=_NS),
        scratch_types=[
            pltpu.VMEM((_UPW * _T,), jnp.int32),
            pltpu.VMEM((_UPW * _T, _SUB), jnp.float32),
            pltpu.SemaphoreType.DMA,
        ],
        compiler_params=pltpu.CompilerParams(use_tc_tiling_on_sc=False),
    )


def kernel(x, W):
    B, T, D = x.shape
    idxg, err = _distances(x, W)
    table = W.reshape(_S * _K, _SUB)
    idx_flat = idxg.reshape(_B * _S * _T)
    quant = _gather_kernel()(table, idx_flat)
    return quant.reshape(B, T, D), err[0, 0]
